# Initial kernel scaffold; baseline (speedup 1.0000x reference)
import jax
import jax.numpy as jnp
from jax.experimental import pallas as pl


def kernel(x_forward, x_backward, indices_fwd, indices_back):
    raise NotImplementedError("write your pallas kernel here")


# speedup vs baseline: 2.5106x; 2.5106x over previous
import functools

import jax
import jax.numpy as jnp
from jax import lax
from jax.experimental import pallas as pl
from jax.experimental.pallas import tpu as pltpu
from jax.experimental.pallas import tpu_sc as plsc

_L = 16


@functools.lru_cache(maxsize=None)
def _make_sc_kernel(B, S, D, NW, K):
    R = B * S
    rows_w = R // NW
    n_chunks = rows_w // K
    wpb = S // rows_w

    mesh = plsc.VectorSubcoreMesh(core_axis_name="c", subcore_axis_name="s")

    @functools.partial(
        pl.kernel,
        mesh=mesh,
        out_type=jax.ShapeDtypeStruct((R, 2 * D), jnp.float32),
        scratch_types=[
            pltpu.VMEM((2 * n_chunks, K), jnp.int32),
            pltpu.VMEM((K, 2 * D), jnp.float32),
            pltpu.SemaphoreType.DMA,
        ],
    )
    def sc_kernel(xf_hbm, xb_hbm, idxf_hbm, idxb_hbm, out_hbm, idx_v, buf_v, gsem):
        wid = lax.axis_index("s") * 2 + lax.axis_index("c")
        row_base = wid * rows_w
        boff = (wid // wpb) * S

        pltpu.sync_copy(idxf_hbm.at[wid], idx_v.at[pl.ds(0, n_chunks)])
        pltpu.sync_copy(idxb_hbm.at[wid], idx_v.at[pl.ds(n_chunks, n_chunks)])

        def mask_body(t, carry):
            for h in range(K // _L):
                v = idx_v[t, pl.ds(h * _L, _L)]
                v = jnp.where(v >= 2, v, 0) + boff
                idx_v[t, pl.ds(h * _L, _L)] = v
            return carry

        lax.fori_loop(0, 2 * n_chunks, mask_body, 0)

        def chunk_body(c, carry):
            cf = pltpu.async_copy(
                xf_hbm.at[idx_v.at[c]], buf_v.at[:, pl.ds(0, D)], gsem)
            cb = pltpu.async_copy(
                xb_hbm.at[idx_v.at[n_chunks + c]], buf_v.at[:, pl.ds(D, D)], gsem)
            cf.wait()
            cb.wait()
            pltpu.sync_copy(buf_v, out_hbm.at[pl.ds(row_base + c * K, K)])
            return carry

        lax.fori_loop(0, n_chunks, chunk_body, 0)

    return sc_kernel


def kernel(x_forward, x_backward, indices_fwd, indices_back):
    B, S, D = x_forward.shape
    NW, K = 32, 64
    R = B * S
    rows_w = R // NW
    n_chunks = rows_w // K

    xf = x_forward.reshape(R, D)
    xb = x_backward.reshape(R, D)
    idf = indices_fwd.astype(jnp.int32).reshape(NW, n_chunks, K)
    idb = indices_back.astype(jnp.int32).reshape(NW, n_chunks, K)

    out = _make_sc_kernel(B, S, D, NW, K)(xf, xb, idf, idb)
    return out.reshape(B, S, 2 * D)


# speedup vs baseline: 2.5758x; 1.0260x over previous
import functools

import jax
import jax.numpy as jnp
from jax import lax
from jax.experimental import pallas as pl
from jax.experimental.pallas import tpu as pltpu
from jax.experimental.pallas import tpu_sc as plsc

_L = 16


@functools.lru_cache(maxsize=None)
def _make_sc_kernel(B, S, D, NW, K):
    R = B * S
    rows_w = R // NW
    n_chunks = rows_w // K
    wpb = S // rows_w

    mesh = plsc.VectorSubcoreMesh(core_axis_name="c", subcore_axis_name="s")

    @functools.partial(
        pl.kernel,
        mesh=mesh,
        out_type=jax.ShapeDtypeStruct((R, 2 * D), jnp.float32),
        scratch_types=[
            pltpu.VMEM((2 * n_chunks, K), jnp.int32),
            pltpu.VMEM((2, K, 2 * D), jnp.float32),
            pltpu.SemaphoreType.DMA,
            pltpu.SemaphoreType.DMA,
            pltpu.SemaphoreType.DMA,
            pltpu.SemaphoreType.DMA,
        ],
    )
    def sc_kernel(xf_hbm, xb_hbm, idxf_hbm, idxb_hbm, out_hbm,
                  idx_v, buf_v, gsem0, gsem1, wsem0, wsem1):
        gsems = (gsem0, gsem1)
        wsems = (wsem0, wsem1)
        wid = lax.axis_index("s") * 2 + lax.axis_index("c")
        row_base = wid * rows_w
        boff = (wid // wpb) * S

        pltpu.sync_copy(idxf_hbm.at[wid], idx_v.at[pl.ds(0, n_chunks)])
        pltpu.sync_copy(idxb_hbm.at[wid], idx_v.at[pl.ds(n_chunks, n_chunks)])

        def mask_body(t, carry):
            for h in range(K // _L):
                v = idx_v[t, pl.ds(h * _L, _L)]
                v = jnp.where(v >= 2, v, 0) + boff
                idx_v[t, pl.ds(h * _L, _L)] = v
            return carry

        lax.fori_loop(0, 2 * n_chunks, mask_body, 0)

        def gather_copies(c, s):
            cf = pltpu.make_async_copy(
                xf_hbm.at[idx_v.at[c]], buf_v.at[s, :, pl.ds(0, D)], gsems[s])
            cb = pltpu.make_async_copy(
                xb_hbm.at[idx_v.at[n_chunks + c]], buf_v.at[s, :, pl.ds(D, D)],
                gsems[s])
            return cf, cb

        def write_copy(c, s):
            return pltpu.make_async_copy(
                buf_v.at[s], out_hbm.at[pl.ds(row_base + c * K, K)], wsems[s])

        def fire_gather(c, s):
            cf, cb = gather_copies(c, s)
            cf.start()
            cb.start()

        fire_gather(0, 0)

        def chunk_body(i, carry):
            for s in range(2):
                c = 2 * i + s
                cf, cb = gather_copies(c, s)
                cf.wait()
                cb.wait()
                write_copy(c, s).start()

                @pl.when(c + 1 < n_chunks)
                def _fire_next():
                    @pl.when(c >= 1)
                    def _drain_prev_write():
                        write_copy(c - 1, 1 - s).wait()
                    fire_gather(c + 1, 1 - s)
            return carry

        lax.fori_loop(0, n_chunks // 2, chunk_body, 0)
        write_copy(n_chunks - 1, 1).wait()

    return sc_kernel


def kernel(x_forward, x_backward, indices_fwd, indices_back):
    B, S, D = x_forward.shape
    NW, K = 32, 32
    R = B * S
    rows_w = R // NW
    n_chunks = rows_w // K

    xf = x_forward.reshape(R, D)
    xb = x_backward.reshape(R, D)
    idf = indices_fwd.astype(jnp.int32).reshape(NW, n_chunks, K)
    idb = indices_back.astype(jnp.int32).reshape(NW, n_chunks, K)

    out = _make_sc_kernel(B, S, D, NW, K)(xf, xb, idf, idb)
    return out.reshape(B, S, 2 * D)
